# baseline (speedup 1.0000x reference)
import jax
import jax.numpy as jnp
from jax import lax
from jax.experimental import pallas as pl
from jax.experimental.pallas import tpu as pltpu
from jax.experimental.pallas import tpu_sc as plsc

N = 10000
E = 160000
SDIM = 128
VDIM = 32
EDIM = 16
L = 5
NG = 64
V3 = 3 * VDIM

NC = 2
NS = 16
NW = NC * NS
NPT = N // NS

f32 = jnp.float32
i32 = jnp.int32


def _mesh():
    return plsc.VectorSubcoreMesh(core_axis_name="c", subcore_axis_name="s")


def _silu16(x):
    return x / (1.0 + jnp.exp(-x))


def _edge_chunks(wid, ce, chunk):
    nb_full = E // (NW * ce)
    tail_base = NW * nb_full * ce
    n_tail = (E - tail_base) // ce

    def loop_body(i, carry):
        chunk(wid * nb_full * ce + i * ce)
        return carry

    lax.fori_loop(0, nb_full, loop_body, 0)
    if n_tail:
        @pl.when(wid < n_tail)
        def _():
            chunk(tail_base + wid * ce)


def _sc_gather16(p16, src, dst, z16):
    CE = 128

    def body(p16_h, src_h, dst_h, z_h, psrc_h, pdst_h, cntp_h,
             sidx, didx, bs_, bd_, ones, acc, sem):
        cid = lax.axis_index("c")
        sid = lax.axis_index("s")
        wid = sid * NC + cid

        pltpu.sync_copy(z_h.at[pl.ds(sid * NPT, NPT)],
                        acc.at[pl.ds(sid * NPT, NPT)])

        def fill_body(e, carry):
            ones[e, :] = jnp.full((16,), 1.0, f32)
            return carry

        lax.fori_loop(0, CE, fill_body, 0)
        plsc.subcore_barrier()

        def chunk(base):
            pltpu.sync_copy(src_h.at[pl.ds(base, CE)], sidx)
            pltpu.sync_copy(dst_h.at[pl.ds(base, CE)], didx)
            pltpu.async_copy(p16_h.at[sidx], bs_, sem).wait()
            pltpu.async_copy(p16_h.at[didx], bd_, sem).wait()
            pltpu.sync_copy(bs_, psrc_h.at[pl.ds(base, CE)])
            pltpu.sync_copy(bd_, pdst_h.at[pl.ds(base, CE)])
            pltpu.sync_copy(ones, acc.at[didx], add=True)

        _edge_chunks(wid, CE, chunk)

        plsc.subcore_barrier()
        pltpu.sync_copy(acc.at[pl.ds(sid * NPT, NPT)],
                        cntp_h.at[cid, pl.ds(sid * NPT, NPT)])

    out = pl.kernel(
        body,
        out_type=(
            jax.ShapeDtypeStruct((E, 16), f32),
            jax.ShapeDtypeStruct((E, 16), f32),
            jax.ShapeDtypeStruct((NC, N, 16), f32),
        ),
        mesh=_mesh(),
        compiler_params=pltpu.CompilerParams(use_tc_tiling_on_sc=False),
        scratch_types=[
            pltpu.VMEM((CE,), i32),
            pltpu.VMEM((CE,), i32),
            pltpu.VMEM((CE, 16), f32),
            pltpu.VMEM((CE, 16), f32),
            pltpu.VMEM((CE, 16), f32),
            pltpu.VMEM_SHARED((N, 16), f32),
            pltpu.SemaphoreType.DMA,
        ],
    )
    return out(p16, src, dst, z16)


def _sc_pass1(p1, p2, eproj, src, dst, z128):
    CE = 64

    def body(p1_h, p2_h, ep_h, src_h, dst_h, z_h, msp_h, gg_h,
             sidx, didx, ab, bb_, msb, ggb, acc, sem):
        cid = lax.axis_index("c")
        sid = lax.axis_index("s")
        wid = sid * NC + cid

        pltpu.sync_copy(z_h.at[pl.ds(sid * NPT, NPT)],
                        acc.at[pl.ds(sid * NPT, NPT)])
        plsc.subcore_barrier()

        def chunk(base):
            pltpu.sync_copy(src_h.at[pl.ds(base, CE)], sidx)
            pltpu.sync_copy(dst_h.at[pl.ds(base, CE)], didx)
            pltpu.async_copy(p1_h.at[sidx], ab, sem).wait()
            pltpu.async_copy(p2_h.at[didx], bb_, sem).wait()

            def add_body(e, carry):
                for j in range(12):
                    sl = pl.ds(16 * j, 16)
                    ab[e, sl] = ab[e, sl] + bb_[e, sl]
                return carry

            lax.fori_loop(0, CE, add_body, 0)
            pltpu.sync_copy(ep_h.at[pl.ds(base, CE)], bb_)

            def silu_body(e, carry):
                for j in range(12):
                    sl = pl.ds(16 * j, 16)
                    y = _silu16(ab[e, sl] + bb_[e, sl])
                    if j < 8:
                        msb[e, sl] = y
                    else:
                        ggb[e, pl.ds(16 * (j - 8), 16)] = y
                return carry

            lax.fori_loop(0, CE, silu_body, 0)
            pltpu.sync_copy(msb, acc.at[didx], add=True)
            pltpu.sync_copy(ggb, gg_h.at[pl.ds(base, CE)])

        _edge_chunks(wid, CE, chunk)

        plsc.subcore_barrier()
        pltpu.sync_copy(acc.at[pl.ds(sid * NPT, NPT)],
                        msp_h.at[cid, pl.ds(sid * NPT, NPT)])

    out = pl.kernel(
        body,
        out_type=(
            jax.ShapeDtypeStruct((NC, N, SDIM), f32),
            jax.ShapeDtypeStruct((E, 2 * VDIM), f32),
        ),
        mesh=_mesh(),
        compiler_params=pltpu.CompilerParams(use_tc_tiling_on_sc=False),
        scratch_types=[
            pltpu.VMEM((CE,), i32),
            pltpu.VMEM((CE,), i32),
            pltpu.VMEM((CE, 192), f32),
            pltpu.VMEM((CE, 192), f32),
            pltpu.VMEM((CE, SDIM), f32),
            pltpu.VMEM((CE, 2 * VDIM), f32),
            pltpu.VMEM_SHARED((N, SDIM), f32),
            pltpu.SemaphoreType.DMA,
        ],
    )
    return out(p1, p2, eproj, src, dst, z128)


def _sc_pass2(vcur, gvgr, eg, src, dst, z96):
    CE = 128

    def body(v_h, gg_h, eg_h, src_h, dst_h, z_h, mvp_h,
             sidx, didx, vgb, ggb, egb, mvb, acc, sem):
        cid = lax.axis_index("c")
        sid = lax.axis_index("s")
        wid = sid * NC + cid

        pltpu.sync_copy(z_h.at[pl.ds(sid * NPT, NPT)],
                        acc.at[pl.ds(sid * NPT, NPT)])
        plsc.subcore_barrier()

        def chunk(base):
            pltpu.sync_copy(src_h.at[pl.ds(base, CE)], sidx)
            pltpu.sync_copy(dst_h.at[pl.ds(base, CE)], didx)
            pltpu.async_copy(v_h.at[sidx], vgb, sem).wait()
            pltpu.sync_copy(gg_h.at[pl.ds(base, CE)], ggb)
            pltpu.sync_copy(eg_h.at[pl.ds(base, CE)], egb)

            def mv_body(e, carry):
                gv0 = ggb[e, pl.ds(0, 16)]
                gv1 = ggb[e, pl.ds(16, 16)]
                gr0 = ggb[e, pl.ds(32, 16)]
                gr1 = ggb[e, pl.ds(48, 16)]
                egv = egb[e, pl.ds(0, 16)]
                for k in range(3):
                    rnk = jnp.full((16,), egv[k], f32)
                    s0 = pl.ds(32 * k, 16)
                    s1 = pl.ds(32 * k + 16, 16)
                    mvb[e, s0] = gv0 * vgb[e, s0] + gr0 * rnk
                    mvb[e, s1] = gv1 * vgb[e, s1] + gr1 * rnk
                return carry

            lax.fori_loop(0, CE, mv_body, 0)
            pltpu.sync_copy(mvb, acc.at[didx], add=True)

        _edge_chunks(wid, CE, chunk)

        plsc.subcore_barrier()
        pltpu.sync_copy(acc.at[pl.ds(sid * NPT, NPT)],
                        mvp_h.at[cid, pl.ds(sid * NPT, NPT)])

    out = pl.kernel(
        body,
        out_type=jax.ShapeDtypeStruct((NC, N, V3), f32),
        mesh=_mesh(),
        compiler_params=pltpu.CompilerParams(use_tc_tiling_on_sc=False),
        scratch_types=[
            pltpu.VMEM((CE,), i32),
            pltpu.VMEM((CE,), i32),
            pltpu.VMEM((CE, V3), f32),
            pltpu.VMEM((CE, 2 * VDIM), f32),
            pltpu.VMEM((CE, 16), f32),
            pltpu.VMEM((CE, V3), f32),
            pltpu.VMEM_SHARED((N, V3), f32),
            pltpu.SemaphoreType.DMA,
        ],
    )
    return out(vcur, gvgr, eg, src, dst, z96)


def _tc_pre_pos(pos, batch_col, batch_row):
    def body(pos_ref, bc_ref, br_ref, out_ref):
        pos_v = pos_ref[...]
        giota_row = lax.broadcasted_iota(i32, (1, NG), 1)
        onehot = (bc_ref[...] == giota_row).astype(f32)
        giota_col = lax.broadcasted_iota(i32, (NG, 1), 0)
        onehot_t = (giota_col == br_ref[...]).astype(f32)
        sums = jnp.dot(onehot_t, pos_v, preferred_element_type=f32)
        cnts = jnp.sum(onehot_t, axis=1, keepdims=True)
        means = sums / jnp.maximum(cnts, 1.0)
        cent = pos_v - jnp.dot(onehot, means, preferred_element_type=f32)
        n2 = jnp.sum(cent * cent, axis=1, keepdims=True)
        nrm = jnp.maximum(jnp.sqrt(n2), 1e-12)
        pn = cent / nrm
        out_ref[:, 0:3] = cent
        out_ref[:, 3:6] = pn
        out_ref[:, 6:16] = jnp.zeros((N, 10), f32)

    return pl.pallas_call(
        body,
        out_shape=jax.ShapeDtypeStruct((N, 16), f32),
    )(pos, batch_col, batch_row)


def _tc_matmul_bias(x, w, b, bn):
    m, k = x.shape
    _, n = w.shape
    grid = (m + bn - 1) // bn

    def body(x_ref, w_ref, b_ref, o_ref):
        o_ref[...] = jnp.dot(x_ref[...], w_ref[...],
                             preferred_element_type=f32) + b_ref[...]

    return pl.pallas_call(
        body,
        grid=(grid,),
        in_specs=[
            pl.BlockSpec((bn, k), lambda i: (i, 0)),
            pl.BlockSpec((k, n), lambda i: (0, 0)),
            pl.BlockSpec((1, n), lambda i: (0, 0)),
        ],
        out_specs=pl.BlockSpec((bn, n), lambda i: (i, 0)),
        out_shape=jax.ShapeDtypeStruct((m, n), f32),
    )(x, w, b)


def _tc_geom(psrc, pdst, ea8, wb8, bb_row):
    bn = 1000
    grid = E // bn

    def body(ps_ref, pd_ref, ea_ref, wb_ref, bbr_ref, eg_ref, ec_ref):
        ps = ps_ref[:, 0:3]
        pd = pd_ref[:, 0:3]
        pns = ps_ref[:, 3:6]
        pnd = pd_ref[:, 3:6]
        r = pd - ps
        d2 = jnp.sum(r * r, axis=1, keepdims=True)
        d = jnp.sqrt(jnp.maximum(d2, 1e-6))
        a = jnp.sum(pnd * pns, axis=1, keepdims=True)
        rn = r / (d + 1.0)
        ea = jnp.dot(ea_ref[...], wb_ref[...],
                     preferred_element_type=f32) + bbr_ref[...]
        eg_ref[:, 0:3] = rn
        eg_ref[:, 3:4] = d
        eg_ref[:, 4:5] = a
        eg_ref[:, 5:16] = jnp.zeros((bn, 11), f32)
        ec_ref[:, 0:16] = ea
        ec_ref[:, 16:17] = d
        ec_ref[:, 17:18] = a
        ec_ref[:, 18:24] = jnp.zeros((bn, 6), f32)

    return pl.pallas_call(
        body,
        grid=(grid,),
        in_specs=[
            pl.BlockSpec((bn, 16), lambda i: (i, 0)),
            pl.BlockSpec((bn, 16), lambda i: (i, 0)),
            pl.BlockSpec((bn, 8), lambda i: (i, 0)),
            pl.BlockSpec((8, 16), lambda i: (0, 0)),
            pl.BlockSpec((1, 16), lambda i: (0, 0)),
        ],
        out_specs=[
            pl.BlockSpec((bn, 16), lambda i: (i, 0)),
            pl.BlockSpec((bn, 24), lambda i: (i, 0)),
        ],
        out_shape=[
            jax.ShapeDtypeStruct((E, 16), f32),
            jax.ShapeDtypeStruct((E, 24), f32),
        ],
    )(psrc, pdst, ea8, wb8, bb_row)


def _tc_proj12(s, w12):
    bn = 1000
    grid = N // bn

    def body(s_ref, w_ref, p1_ref, p2_ref):
        p12 = jnp.dot(s_ref[...], w_ref[...], preferred_element_type=f32)
        p1_ref[...] = p12[:, 0:192]
        p2_ref[...] = p12[:, 192:384]

    return pl.pallas_call(
        body,
        grid=(grid,),
        in_specs=[
            pl.BlockSpec((bn, SDIM), lambda i: (i, 0)),
            pl.BlockSpec((SDIM, 384), lambda i: (0, 0)),
        ],
        out_specs=[
            pl.BlockSpec((bn, 192), lambda i: (i, 0)),
            pl.BlockSpec((bn, 192), lambda i: (i, 0)),
        ],
        out_shape=[
            jax.ShapeDtypeStruct((N, 192), f32),
            jax.ShapeDtypeStruct((N, 192), f32),
        ],
    )(s, w12)


def _tc_update(s, v, ms0, ms1, mv0, mv1, cp0, cp1, ws, bs_row):
    bn = 1000
    grid = N // bn

    def body(s_ref, v_ref, a_ref, b_ref, c_ref, d_ref, e_ref, f_ref,
             w_ref, br_ref, so_ref, vo_ref):
        cnt = e_ref[:, 0:1] + f_ref[:, 0:1]
        invc = 1.0 / jnp.maximum(cnt, 1.0)
        sm = (a_ref[...] + b_ref[...]) * invc
        z = jnp.dot(sm, w_ref[...], preferred_element_type=f32) + br_ref[...]
        so_ref[...] = s_ref[...] + z * jax.nn.sigmoid(z)
        vo_ref[...] = v_ref[...] + (c_ref[...] + d_ref[...]) * invc

    return pl.pallas_call(
        body,
        grid=(grid,),
        in_specs=[
            pl.BlockSpec((bn, SDIM), lambda i: (i, 0)),
            pl.BlockSpec((bn, V3), lambda i: (i, 0)),
            pl.BlockSpec((bn, SDIM), lambda i: (i, 0)),
            pl.BlockSpec((bn, SDIM), lambda i: (i, 0)),
            pl.BlockSpec((bn, V3), lambda i: (i, 0)),
            pl.BlockSpec((bn, V3), lambda i: (i, 0)),
            pl.BlockSpec((bn, 16), lambda i: (i, 0)),
            pl.BlockSpec((bn, 16), lambda i: (i, 0)),
            pl.BlockSpec((SDIM, SDIM), lambda i: (0, 0)),
            pl.BlockSpec((1, SDIM), lambda i: (0, 0)),
        ],
        out_specs=[
            pl.BlockSpec((bn, SDIM), lambda i: (i, 0)),
            pl.BlockSpec((bn, V3), lambda i: (i, 0)),
        ],
        out_shape=[
            jax.ShapeDtypeStruct((N, SDIM), f32),
            jax.ShapeDtypeStruct((N, V3), f32),
        ],
    )(s, v, ms0, ms1, mv0, mv1, cp0, cp1, ws, bs_row)


def kernel(x, pos, edge_index, edge_attr, batch, Wa, ba, Wb, bb, Wm, bm, Ws, bs):
    src = edge_index[0]
    dst = edge_index[1]
    batch_col = batch.reshape(N, 1)
    batch_row = batch.reshape(1, N)
    ea8 = jnp.pad(edge_attr, ((0, 0), (0, 3)))
    wb8 = jnp.pad(Wb, ((0, 3), (0, 0)))
    bb_row = bb.reshape(1, EDIM)
    w12 = jnp.concatenate([Wm[:, :SDIM, :], Wm[:, SDIM:2 * SDIM, :]], axis=-1)
    wme = jnp.pad(Wm[:, 2 * SDIM:, :], ((0, 0), (0, 6), (0, 0)))
    z16 = jnp.zeros((N, 16), f32)
    z128 = jnp.zeros((N, SDIM), f32)
    z96 = jnp.zeros((N, V3), f32)

    p16 = _tc_pre_pos(pos, batch_col, batch_row)
    psrc, pdst, cntp = _sc_gather16(p16, src, dst, z16)
    eg, ec = _tc_geom(psrc, pdst, ea8, wb8, bb_row)
    s = _tc_matmul_bias(x, Wa, ba.reshape(1, SDIM), 1000)

    v = z96
    for l in range(L):
        p1, p2 = _tc_proj12(s, w12[l])
        eproj = _tc_matmul_bias(ec, wme[l], bm[l].reshape(1, 192), 1000)
        msp, gvgr = _sc_pass1(p1, p2, eproj, src, dst, z128)
        mvp = _sc_pass2(v, gvgr, eg, src, dst, z96)
        s, v = _tc_update(s, v, msp[0], msp[1], mvp[0], mvp[1],
                          cntp[0], cntp[1], Ws[l], bs[l].reshape(1, SDIM))
    return s


# speedup vs baseline: 1.8363x; 1.8363x over previous
import jax
import jax.numpy as jnp
from jax import lax
from jax.experimental import pallas as pl
from jax.experimental.pallas import tpu as pltpu
from jax.experimental.pallas import tpu_sc as plsc

N = 10000
E = 160000
SDIM = 128
VDIM = 32
EDIM = 16
L = 5
NG = 64
V3 = 3 * VDIM

NC = 2
NS = 16
NW = NC * NS
NPT = N // NS

f32 = jnp.float32
i32 = jnp.int32


def _mesh():
    return plsc.VectorSubcoreMesh(core_axis_name="c", subcore_axis_name="s")


def _silu16(x):
    return x / (1.0 + jnp.exp(-x))


def _edge_chunks(wid, ce, chunk):
    nb_full = E // (NW * ce)
    tail_base = NW * nb_full * ce
    n_tail = (E - tail_base) // ce

    def loop_body(i, carry):
        chunk(wid * nb_full * ce + i * ce)
        return carry

    lax.fori_loop(0, nb_full, loop_body, 0)
    if n_tail:
        @pl.when(wid < n_tail)
        def _():
            chunk(tail_base + wid * ce)


def _sc_gather16(p16, src, dst, z16):
    CE = 128

    def body(p16_h, src_h, dst_h, z_h, psrc_h, pdst_h, cntp_h,
             sidx, didx, bs_, bd_, ones, acc, sem):
        cid = lax.axis_index("c")
        sid = lax.axis_index("s")
        wid = sid * NC + cid

        pltpu.sync_copy(z_h.at[pl.ds(sid * NPT, NPT)],
                        acc.at[pl.ds(sid * NPT, NPT)])

        def fill_body(e, carry):
            ones[e, :] = jnp.full((16,), 1.0, f32)
            return carry

        lax.fori_loop(0, CE, fill_body, 0)
        plsc.subcore_barrier()

        def chunk(base):
            pltpu.sync_copy(src_h.at[pl.ds(base, CE)], sidx)
            pltpu.sync_copy(dst_h.at[pl.ds(base, CE)], didx)
            pltpu.async_copy(p16_h.at[sidx], bs_, sem).wait()
            pltpu.async_copy(p16_h.at[didx], bd_, sem).wait()
            pltpu.sync_copy(bs_, psrc_h.at[pl.ds(base, CE)])
            pltpu.sync_copy(bd_, pdst_h.at[pl.ds(base, CE)])
            pltpu.sync_copy(ones, acc.at[didx], add=True)

        _edge_chunks(wid, CE, chunk)

        plsc.subcore_barrier()
        pltpu.sync_copy(acc.at[pl.ds(sid * NPT, NPT)],
                        cntp_h.at[cid, pl.ds(sid * NPT, NPT)])

    out = pl.kernel(
        body,
        out_type=(
            jax.ShapeDtypeStruct((E, 16), f32),
            jax.ShapeDtypeStruct((E, 16), f32),
            jax.ShapeDtypeStruct((NC, N, 16), f32),
        ),
        mesh=_mesh(),
        compiler_params=pltpu.CompilerParams(use_tc_tiling_on_sc=False),
        scratch_types=[
            pltpu.VMEM((CE,), i32),
            pltpu.VMEM((CE,), i32),
            pltpu.VMEM((CE, 16), f32),
            pltpu.VMEM((CE, 16), f32),
            pltpu.VMEM((CE, 16), f32),
            pltpu.VMEM_SHARED((N, 16), f32),
            pltpu.SemaphoreType.DMA,
        ],
    )
    return out(p16, src, dst, z16)


CEB = 128
NBW = (E // CEB) // NW
NTAIL = E // CEB - NW * NBW


def _sc_gather3(p1, p2, vcur, src2, dst2):
    def body(p1_h, p2_h, v_h, s2_h, d2_h, g1_h, g2_h, vg_h,
             sidx2, didx2, tsi, tdi, a0, a1, b0, b1, c0,
             gsem0, gsem1, wsem0, wsem1):
        cid = lax.axis_index("c")
        sid = lax.axis_index("s")
        wid = sid * NC + cid
        row0 = wid * NBW
        pltpu.sync_copy(s2_h.at[pl.ds(row0, NBW)], sidx2)
        pltpu.sync_copy(d2_h.at[pl.ds(row0, NBW)], didx2)
        ab = (a0, a1)
        bb_ = (b0, b1)
        gsem = (gsem0, gsem1)
        wsem = (wsem0, wsem1)
        gds = [None, None]
        wds = [None, None]

        def issue_gathers(i):
            sl = i % 2
            gds[sl] = (
                pltpu.async_copy(p1_h.at[sidx2.at[i]], ab[sl], gsem[sl]),
                pltpu.async_copy(p2_h.at[didx2.at[i]], bb_[sl], gsem[sl]),
            )

        issue_gathers(0)
        for i in range(NBW):
            sl = i % 2
            vds = pltpu.async_copy(v_h.at[sidx2.at[i]], c0, gsem[sl])
            if i + 1 < NBW:
                osl = (i + 1) % 2
                if wds[osl] is not None:
                    for dsc in wds[osl]:
                        dsc.wait()
                    wds[osl] = None
                issue_gathers(i + 1)
            for dsc in gds[sl]:
                dsc.wait()
            vds.wait()
            base = (row0 + i) * CEB
            wds[sl] = (
                pltpu.async_copy(ab[sl], g1_h.at[pl.ds(base, CEB)], wsem[sl]),
                pltpu.async_copy(bb_[sl], g2_h.at[pl.ds(base, CEB)], wsem[sl]),
            )
            pltpu.sync_copy(c0, vg_h.at[pl.ds(base, CEB)])
        for sl in range(2):
            if wds[sl] is not None:
                for dsc in wds[sl]:
                    dsc.wait()

        @pl.when(wid < NTAIL)
        def _():
            trow = NW * NBW + wid
            pltpu.sync_copy(s2_h.at[pl.ds(trow, 1)], tsi)
            pltpu.sync_copy(d2_h.at[pl.ds(trow, 1)], tdi)
            pltpu.async_copy(p1_h.at[tsi.at[0]], a0, gsem0).wait()
            pltpu.async_copy(p2_h.at[tdi.at[0]], b0, gsem0).wait()
            pltpu.async_copy(v_h.at[tsi.at[0]], c0, gsem0).wait()
            tb = trow * CEB
            pltpu.sync_copy(a0, g1_h.at[pl.ds(tb, CEB)])
            pltpu.sync_copy(b0, g2_h.at[pl.ds(tb, CEB)])
            pltpu.sync_copy(c0, vg_h.at[pl.ds(tb, CEB)])

    out = pl.kernel(
        body,
        out_type=(
            jax.ShapeDtypeStruct((E, 192), f32),
            jax.ShapeDtypeStruct((E, 192), f32),
            jax.ShapeDtypeStruct((E, V3), f32),
        ),
        mesh=_mesh(),
        compiler_params=pltpu.CompilerParams(use_tc_tiling_on_sc=False),
        scratch_types=[
            pltpu.VMEM((NBW, CEB), i32),
            pltpu.VMEM((NBW, CEB), i32),
            pltpu.VMEM((1, CEB), i32),
            pltpu.VMEM((1, CEB), i32),
            pltpu.VMEM((CEB, 192), f32),
            pltpu.VMEM((CEB, 192), f32),
            pltpu.VMEM((CEB, 192), f32),
            pltpu.VMEM((CEB, 192), f32),
            pltpu.VMEM((CEB, V3), f32),
            pltpu.SemaphoreType.DMA,
            pltpu.SemaphoreType.DMA,
            pltpu.SemaphoreType.DMA,
            pltpu.SemaphoreType.DMA,
        ],
    )
    return out(p1, p2, vcur, src2, dst2)


def _sc_scatter(rows, dst2, zacc):
    W = rows.shape[1]

    def body(rows_h, d2_h, z_h, part_h, didx2, tdi, b0, b1, acc,
             rsem0, rsem1):
        cid = lax.axis_index("c")
        sid = lax.axis_index("s")
        wid = sid * NC + cid
        row0 = wid * NBW
        pltpu.sync_copy(z_h.at[pl.ds(sid * NPT, NPT)],
                        acc.at[pl.ds(sid * NPT, NPT)])
        pltpu.sync_copy(d2_h.at[pl.ds(row0, NBW)], didx2)
        plsc.subcore_barrier()
        bufs = (b0, b1)
        rsem = (rsem0, rsem1)
        rds = [None, None]

        def issue_read(i):
            sl = i % 2
            rds[sl] = pltpu.async_copy(
                rows_h.at[pl.ds((row0 + i) * CEB, CEB)], bufs[sl], rsem[sl])

        issue_read(0)
        for i in range(NBW):
            sl = i % 2
            if i + 1 < NBW:
                issue_read(i + 1)
            rds[sl].wait()
            pltpu.sync_copy(bufs[sl], acc.at[didx2.at[i]], add=True)

        @pl.when(wid < NTAIL)
        def _():
            trow = NW * NBW + wid
            pltpu.sync_copy(d2_h.at[pl.ds(trow, 1)], tdi)
            pltpu.sync_copy(rows_h.at[pl.ds(trow * CEB, CEB)], b0)
            pltpu.sync_copy(b0, acc.at[tdi.at[0]], add=True)

        plsc.subcore_barrier()
        pltpu.sync_copy(acc.at[pl.ds(sid * NPT, NPT)],
                        part_h.at[cid, pl.ds(sid * NPT, NPT)])

    out = pl.kernel(
        body,
        out_type=jax.ShapeDtypeStruct((NC, N, W), f32),
        mesh=_mesh(),
        compiler_params=pltpu.CompilerParams(use_tc_tiling_on_sc=False),
        scratch_types=[
            pltpu.VMEM((NBW, CEB), i32),
            pltpu.VMEM((1, CEB), i32),
            pltpu.VMEM((CEB, W), f32),
            pltpu.VMEM((CEB, W), f32),
            pltpu.VMEM_SHARED((N, W), f32),
            pltpu.SemaphoreType.DMA,
            pltpu.SemaphoreType.DMA,
        ],
    )
    return out(rows, dst2, zacc)


def _tc_pre_pos(pos, batch_col, batch_row):
    def body(pos_ref, bc_ref, br_ref, out_ref):
        pos_v = pos_ref[...]
        giota_row = lax.broadcasted_iota(i32, (1, NG), 1)
        onehot = (bc_ref[...] == giota_row).astype(f32)
        giota_col = lax.broadcasted_iota(i32, (NG, 1), 0)
        onehot_t = (giota_col == br_ref[...]).astype(f32)
        sums = jnp.dot(onehot_t, pos_v, preferred_element_type=f32)
        cnts = jnp.sum(onehot_t, axis=1, keepdims=True)
        means = sums / jnp.maximum(cnts, 1.0)
        cent = pos_v - jnp.dot(onehot, means, preferred_element_type=f32)
        n2 = jnp.sum(cent * cent, axis=1, keepdims=True)
        nrm = jnp.maximum(jnp.sqrt(n2), 1e-12)
        pn = cent / nrm
        out_ref[:, 0:3] = cent
        out_ref[:, 3:6] = pn
        out_ref[:, 6:16] = jnp.zeros((N, 10), f32)

    return pl.pallas_call(
        body,
        out_shape=jax.ShapeDtypeStruct((N, 16), f32),
    )(pos, batch_col, batch_row)


def _tc_matmul_bias(x, w, b, bn):
    m, k = x.shape
    _, n = w.shape
    grid = (m + bn - 1) // bn

    def body(x_ref, w_ref, b_ref, o_ref):
        o_ref[...] = jnp.dot(x_ref[...], w_ref[...],
                             preferred_element_type=f32) + b_ref[...]

    return pl.pallas_call(
        body,
        grid=(grid,),
        in_specs=[
            pl.BlockSpec((bn, k), lambda i: (i, 0)),
            pl.BlockSpec((k, n), lambda i: (0, 0)),
            pl.BlockSpec((1, n), lambda i: (0, 0)),
        ],
        out_specs=pl.BlockSpec((bn, n), lambda i: (i, 0)),
        out_shape=jax.ShapeDtypeStruct((m, n), f32),
    )(x, w, b)


def _tc_geom(psrc, pdst, ea8, wb8, bb_row):
    bn = 1000
    grid = E // bn

    def body(ps_ref, pd_ref, ea_ref, wb_ref, bbr_ref, eg_ref, ec_ref):
        ps = ps_ref[:, 0:3]
        pd = pd_ref[:, 0:3]
        pns = ps_ref[:, 3:6]
        pnd = pd_ref[:, 3:6]
        r = pd - ps
        d2 = jnp.sum(r * r, axis=1, keepdims=True)
        d = jnp.sqrt(jnp.maximum(d2, 1e-6))
        a = jnp.sum(pnd * pns, axis=1, keepdims=True)
        rn = r / (d + 1.0)
        ea = jnp.dot(ea_ref[...], wb_ref[...],
                     preferred_element_type=f32) + bbr_ref[...]
        eg_ref[:, 0:3] = rn
        eg_ref[:, 3:4] = d
        eg_ref[:, 4:5] = a
        eg_ref[:, 5:16] = jnp.zeros((bn, 11), f32)
        ec_ref[:, 0:16] = ea
        ec_ref[:, 16:17] = d
        ec_ref[:, 17:18] = a
        ec_ref[:, 18:24] = jnp.zeros((bn, 6), f32)

    return pl.pallas_call(
        body,
        grid=(grid,),
        in_specs=[
            pl.BlockSpec((bn, 16), lambda i: (i, 0)),
            pl.BlockSpec((bn, 16), lambda i: (i, 0)),
            pl.BlockSpec((bn, 8), lambda i: (i, 0)),
            pl.BlockSpec((8, 16), lambda i: (0, 0)),
            pl.BlockSpec((1, 16), lambda i: (0, 0)),
        ],
        out_specs=[
            pl.BlockSpec((bn, 16), lambda i: (i, 0)),
            pl.BlockSpec((bn, 24), lambda i: (i, 0)),
        ],
        out_shape=[
            jax.ShapeDtypeStruct((E, 16), f32),
            jax.ShapeDtypeStruct((E, 24), f32),
        ],
    )(psrc, pdst, ea8, wb8, bb_row)


def _tc_proj12(s, w12):
    bn = 1000
    grid = N // bn

    def body(s_ref, w_ref, p1_ref, p2_ref):
        p12 = jnp.dot(s_ref[...], w_ref[...], preferred_element_type=f32)
        p1_ref[...] = p12[:, 0:192]
        p2_ref[...] = p12[:, 192:384]

    return pl.pallas_call(
        body,
        grid=(grid,),
        in_specs=[
            pl.BlockSpec((bn, SDIM), lambda i: (i, 0)),
            pl.BlockSpec((SDIM, 384), lambda i: (0, 0)),
        ],
        out_specs=[
            pl.BlockSpec((bn, 192), lambda i: (i, 0)),
            pl.BlockSpec((bn, 192), lambda i: (i, 0)),
        ],
        out_shape=[
            jax.ShapeDtypeStruct((N, 192), f32),
            jax.ShapeDtypeStruct((N, 192), f32),
        ],
    )(s, w12)


def _tc_edge(g1, g2, vg, ec, eg, wme_l, bm_row):
    bn = 1000
    grid = E // bn

    def body(g1_ref, g2_ref, vg_ref, ec_ref, eg_ref, w_ref, b_ref,
             ms_ref, mv_ref):
        z = (g1_ref[...] + g2_ref[...]
             + jnp.dot(ec_ref[...], w_ref[...], preferred_element_type=f32)
             + b_ref[...])
        h = z * jax.nn.sigmoid(z)
        ms_ref[...] = h[:, 0:SDIM]
        gv = h[:, SDIM:SDIM + VDIM]
        gr = h[:, SDIM + VDIM:]
        for k in range(3):
            sl = slice(VDIM * k, VDIM * (k + 1))
            mv_ref[:, sl] = gv * vg_ref[:, sl] + gr * eg_ref[:, k:k + 1]

    return pl.pallas_call(
        body,
        grid=(grid,),
        in_specs=[
            pl.BlockSpec((bn, 192), lambda i: (i, 0)),
            pl.BlockSpec((bn, 192), lambda i: (i, 0)),
            pl.BlockSpec((bn, V3), lambda i: (i, 0)),
            pl.BlockSpec((bn, 24), lambda i: (i, 0)),
            pl.BlockSpec((bn, 16), lambda i: (i, 0)),
            pl.BlockSpec((24, 192), lambda i: (0, 0)),
            pl.BlockSpec((1, 192), lambda i: (0, 0)),
        ],
        out_specs=[
            pl.BlockSpec((bn, SDIM), lambda i: (i, 0)),
            pl.BlockSpec((bn, V3), lambda i: (i, 0)),
        ],
        out_shape=[
            jax.ShapeDtypeStruct((E, SDIM), f32),
            jax.ShapeDtypeStruct((E, V3), f32),
        ],
    )(g1, g2, vg, ec, eg, wme_l, bm_row)


def _tc_update(s, v, ms0, ms1, mv0, mv1, cp0, cp1, ws, bs_row):
    bn = 1000
    grid = N // bn

    def body(s_ref, v_ref, a_ref, b_ref, c_ref, d_ref, e_ref, f_ref,
             w_ref, br_ref, so_ref, vo_ref):
        cnt = e_ref[:, 0:1] + f_ref[:, 0:1]
        invc = 1.0 / jnp.maximum(cnt, 1.0)
        sm = (a_ref[...] + b_ref[...]) * invc
        z = jnp.dot(sm, w_ref[...], preferred_element_type=f32) + br_ref[...]
        so_ref[...] = s_ref[...] + z * jax.nn.sigmoid(z)
        vo_ref[...] = v_ref[...] + (c_ref[...] + d_ref[...]) * invc

    return pl.pallas_call(
        body,
        grid=(grid,),
        in_specs=[
            pl.BlockSpec((bn, SDIM), lambda i: (i, 0)),
            pl.BlockSpec((bn, V3), lambda i: (i, 0)),
            pl.BlockSpec((bn, SDIM), lambda i: (i, 0)),
            pl.BlockSpec((bn, SDIM), lambda i: (i, 0)),
            pl.BlockSpec((bn, V3), lambda i: (i, 0)),
            pl.BlockSpec((bn, V3), lambda i: (i, 0)),
            pl.BlockSpec((bn, 16), lambda i: (i, 0)),
            pl.BlockSpec((bn, 16), lambda i: (i, 0)),
            pl.BlockSpec((SDIM, SDIM), lambda i: (0, 0)),
            pl.BlockSpec((1, SDIM), lambda i: (0, 0)),
        ],
        out_specs=[
            pl.BlockSpec((bn, SDIM), lambda i: (i, 0)),
            pl.BlockSpec((bn, V3), lambda i: (i, 0)),
        ],
        out_shape=[
            jax.ShapeDtypeStruct((N, SDIM), f32),
            jax.ShapeDtypeStruct((N, V3), f32),
        ],
    )(s, v, ms0, ms1, mv0, mv1, cp0, cp1, ws, bs_row)


def kernel(x, pos, edge_index, edge_attr, batch, Wa, ba, Wb, bb, Wm, bm, Ws, bs):
    src = edge_index[0]
    dst = edge_index[1]
    src2 = src.reshape(E // CEB, CEB)
    dst2 = dst.reshape(E // CEB, CEB)
    batch_col = batch.reshape(N, 1)
    batch_row = batch.reshape(1, N)
    ea8 = jnp.pad(edge_attr, ((0, 0), (0, 3)))
    wb8 = jnp.pad(Wb, ((0, 3), (0, 0)))
    bb_row = bb.reshape(1, EDIM)
    w12 = jnp.concatenate([Wm[:, :SDIM, :], Wm[:, SDIM:2 * SDIM, :]], axis=-1)
    wme = jnp.pad(Wm[:, 2 * SDIM:, :], ((0, 0), (0, 6), (0, 0)))
    z16 = jnp.zeros((N, 16), f32)
    z128 = jnp.zeros((N, SDIM), f32)
    z96 = jnp.zeros((N, V3), f32)

    p16 = _tc_pre_pos(pos, batch_col, batch_row)
    psrc, pdst, cntp = _sc_gather16(p16, src, dst, z16)
    eg, ec = _tc_geom(psrc, pdst, ea8, wb8, bb_row)
    s = _tc_matmul_bias(x, Wa, ba.reshape(1, SDIM), 1000)

    v = z96
    for l in range(L):
        p1, p2 = _tc_proj12(s, w12[l])
        g1, g2, vg = _sc_gather3(p1, p2, v, src2, dst2)
        ms_e, mv_e = _tc_edge(g1, g2, vg, ec, eg, wme[l], bm[l].reshape(1, 192))
        msp = _sc_scatter(ms_e, dst2, z128)
        mvp = _sc_scatter(mv_e, dst2, z96)
        s, v = _tc_update(s, v, msp[0], msp[1], mvp[0], mvp[1],
                          cntp[0], cntp[1], Ws[l], bs[l].reshape(1, SDIM))
    return s
